# baseline (speedup 1.0000x reference)
import jax
import jax.numpy as jnp
from jax import lax
from jax.experimental import pallas as pl
from jax.experimental.pallas import tpu as pltpu

_BATCH = 16
_N = 32
_S = 2048
_L = 2048
_D = 768
_C = 256
_NCH = _L // _C
_W = _C + 8
_G = _C // 8


def _phase1(se_ref, att_ref, bb):
    att = att_ref[bb, :, :]
    starts = se_ref[bb, :, 0].reshape(_N, 1)
    ends = se_ref[bb, :, 1].reshape(_N, 1)
    pos = lax.broadcasted_iota(jnp.int32, (_N, _S), 1)
    m = (pos >= starts) & (pos < ends)
    sums = jnp.sum(jnp.where(m, att, 0.0), axis=1, keepdims=True)
    maxv = jnp.max(sums)
    idx = lax.broadcasted_iota(jnp.int32, (_N, 1), 0)
    best = jnp.min(jnp.where(sums == maxv, idx, _N))
    sel = maxv > 0.0
    is_best = idx == best
    start = jnp.where(sel, jnp.sum(jnp.where(is_best, starts, 0)), 0)
    end = jnp.where(sel, jnp.sum(jnp.where(is_best, ends, 0)), 0)
    return start, end


def _woff(start, lo):
    return pl.multiple_of(
        jnp.minimum((start + lo) // 8 * 8, _S - _W), 8)


def _issue(ctx_hbm, buf_ref, sem, bb, slot, start, end):
    nv = end - start
    for c in range(_NCH):
        lo = c * _C

        @pl.when(nv > lo)
        def _():
            pltpu.make_async_copy(
                ctx_hbm.at[bb, pl.ds(_woff(start, lo), _W), :],
                buf_ref.at[slot, c], sem.at[slot],
            ).start()


def _kern(se_ref, att_ref, ctx_hbm, out_ref, buf_ref, sm_ref, sem):
    b = pl.program_id(0)
    slot = lax.rem(b, 2)
    nslot = lax.rem(b + 1, 2)

    @pl.when(b == 0)
    def _():
        s0, e0 = _phase1(se_ref, att_ref, 0)
        sm_ref[0, 0] = s0
        sm_ref[0, 1] = e0
        _issue(ctx_hbm, buf_ref, sem, 0, 0, s0, e0)

    @pl.when(b + 1 < _BATCH)
    def _():
        s1, e1 = _phase1(se_ref, att_ref, b + 1)
        sm_ref[nslot, 0] = s1
        sm_ref[nslot, 1] = e1
        _issue(ctx_hbm, buf_ref, sem, b + 1, nslot, s1, e1)

    start = sm_ref[slot, 0]
    end = sm_ref[slot, 1]
    nv = end - start
    d8 = lax.rem(start, 8)

    for c in range(_NCH):
        lo = c * _C

        @pl.when(nv > lo)
        def _():
            pltpu.make_async_copy(
                ctx_hbm.at[b, pl.ds(_woff(start, lo), _W), :],
                buf_ref.at[slot, c], sem.at[slot],
            ).wait()

    for c in range(_NCH):
        lo = c * _C

        @pl.when(nv <= lo)
        def _():
            out_ref[0, lo:lo + _C, :] = jnp.zeros((_C, _D), jnp.float32)

        roff_raw = (start + lo) // 8 * 8
        clamped = roff_raw > _S - _W

        def _fast(masked):
            win = buf_ref[slot, c]
            w3 = win.reshape(_W // 8, 8, _D)
            rolled = pltpu.roll(w3, lax.rem(8 - d8, 8), axis=1)
            sub = lax.broadcasted_iota(jnp.int32, (_G, 8, 1), 1)
            rot3 = jnp.where(sub < 8 - d8,
                             rolled[0:_G, :, :], rolled[1:_G + 1, :, :])
            if masked:
                grp = lax.broadcasted_iota(jnp.int32, (_G, 8, 1), 0)
                rot3 = jnp.where(grp * 8 + sub < (nv - lo), rot3, 0.0)
            out_ref[0, lo:lo + _C, :] = rot3.reshape(_C, _D)

        def _slow():
            t = start + lo - _woff(start, lo)
            win = buf_ref[slot, c]
            shift = lax.rem(_W - t, _W)
            rot = pltpu.roll(win, shift, axis=0)
            rows = lax.broadcasted_iota(jnp.int32, (_C, 1), 0)
            out_ref[0, lo:lo + _C, :] = jnp.where(
                rows < (nv - lo), rot[0:_C, :], 0.0)

        full = nv >= lo + _C
        tail = (nv > lo) & (nv < lo + _C)
        pl.when(full & jnp.logical_not(clamped))(lambda: _fast(False))
        pl.when(tail & jnp.logical_not(clamped))(lambda: _fast(True))
        pl.when((nv > lo) & clamped)(_slow)


@jax.jit
def kernel(startends, attention, context):
    att3 = attention.reshape(_BATCH, 1, _S)
    return pl.pallas_call(
        _kern,
        grid=(_BATCH,),
        in_specs=[
            pl.BlockSpec((_BATCH, _N, 2), lambda b: (0, 0, 0)),
            pl.BlockSpec((_BATCH, 1, _S), lambda b: (0, 0, 0)),
            pl.BlockSpec(memory_space=pltpu.MemorySpace.HBM),
        ],
        out_specs=pl.BlockSpec((1, _L, _D), lambda b: (b, 0, 0)),
        out_shape=jax.ShapeDtypeStruct((_BATCH, _L, _D), jnp.float32),
        scratch_shapes=[
            pltpu.VMEM((2, _NCH, _W, _D), jnp.float32),
            pltpu.SMEM((2, 2), jnp.int32),
            pltpu.SemaphoreType.DMA((2,)),
        ],
        compiler_params=pltpu.CompilerParams(
            dimension_semantics=("arbitrary",)),
    )(startends, att3, context)


# speedup vs baseline: 1.0398x; 1.0398x over previous
import jax
import jax.numpy as jnp
from jax import lax
from jax.experimental import pallas as pl
from jax.experimental.pallas import tpu as pltpu

_BATCH = 16
_N = 32
_S = 2048
_L = 2048
_D = 768
_C = 256
_NCH = _L // _C
_W = _C + 8
_G = _C // 8
_BPS = 2
_NSTEP = _BATCH // _BPS


def _phase1(se_ref, att_ref, bb):
    att = att_ref[bb, :, :]
    starts = se_ref[bb, :, 0].reshape(_N, 1)
    ends = se_ref[bb, :, 1].reshape(_N, 1)
    pos = lax.broadcasted_iota(jnp.int32, (_N, _S), 1)
    m = (pos >= starts) & (pos < ends)
    sums = jnp.sum(jnp.where(m, att, 0.0), axis=1, keepdims=True)
    maxv = jnp.max(sums)
    idx = lax.broadcasted_iota(jnp.int32, (_N, 1), 0)
    best = jnp.min(jnp.where(sums == maxv, idx, _N))
    sel = maxv > 0.0
    is_best = idx == best
    start = jnp.where(sel, jnp.sum(jnp.where(is_best, starts, 0)), 0)
    end = jnp.where(sel, jnp.sum(jnp.where(is_best, ends, 0)), 0)
    return start, end


def _woff(start, lo):
    return pl.multiple_of(
        jnp.minimum((start + lo) // 8 * 8, _S - _W), 8)


def _issue(ctx_hbm, buf_ref, sem, bb, slot, i, start, end):
    nv = end - start
    for c in range(_NCH):
        lo = c * _C

        @pl.when(nv > lo)
        def _():
            pltpu.make_async_copy(
                ctx_hbm.at[bb, pl.ds(_woff(start, lo), _W), :],
                buf_ref.at[slot, i, c], sem.at[slot, i],
            ).start()


def _prefetch(se_ref, att_ref, ctx_hbm, buf_ref, sm_ref, sem, g, slot):
    for i in range(_BPS):
        bb = g * _BPS + i
        s, e = _phase1(se_ref, att_ref, bb)
        sm_ref[slot, i, 0] = s
        sm_ref[slot, i, 1] = e
        _issue(ctx_hbm, buf_ref, sem, bb, slot, i, s, e)


def _kern(se_ref, att_ref, ctx_hbm, out_ref, buf_ref, sm_ref, sem):
    g = pl.program_id(0)
    slot = lax.rem(g, 2)
    nslot = lax.rem(g + 1, 2)

    @pl.when(g == 0)
    def _():
        _prefetch(se_ref, att_ref, ctx_hbm, buf_ref, sm_ref, sem, 0, 0)

    @pl.when(g + 1 < _NSTEP)
    def _():
        _prefetch(se_ref, att_ref, ctx_hbm, buf_ref, sm_ref, sem,
                  g + 1, nslot)

    for i in range(_BPS):
        b = g * _BPS + i
        start = sm_ref[slot, i, 0]
        end = sm_ref[slot, i, 1]
        nv = end - start
        d8 = lax.rem(start, 8)

        for c in range(_NCH):
            lo = c * _C

            @pl.when(nv > lo)
            def _():
                pltpu.make_async_copy(
                    ctx_hbm.at[b, pl.ds(_woff(start, lo), _W), :],
                    buf_ref.at[slot, i, c], sem.at[slot, i],
                ).wait()

        for c in range(_NCH):
            lo = c * _C

            @pl.when(nv <= lo)
            def _():
                out_ref[i, lo:lo + _C, :] = jnp.zeros((_C, _D), jnp.float32)

            roff_raw = (start + lo) // 8 * 8
            clamped = roff_raw > _S - _W

            def _fast(masked, i=i, c=c, lo=lo, nv=nv, d8=d8):
                win = buf_ref[slot, i, c]
                w3 = win.reshape(_W // 8, 8, _D)
                rolled = pltpu.roll(w3, lax.rem(8 - d8, 8), axis=1)
                sub = lax.broadcasted_iota(jnp.int32, (_G, 8, 1), 1)
                rot3 = jnp.where(sub < 8 - d8,
                                 rolled[0:_G, :, :], rolled[1:_G + 1, :, :])
                if masked:
                    grp = lax.broadcasted_iota(jnp.int32, (_G, 8, 1), 0)
                    rot3 = jnp.where(grp * 8 + sub < (nv - lo), rot3, 0.0)
                out_ref[i, lo:lo + _C, :] = rot3.reshape(_C, _D)

            def _slow(i=i, c=c, lo=lo, nv=nv, start=start):
                t = start + lo - _woff(start, lo)
                win = buf_ref[slot, i, c]
                shift = lax.rem(_W - t, _W)
                rot = pltpu.roll(win, shift, axis=0)
                rows = lax.broadcasted_iota(jnp.int32, (_C, 1), 0)
                out_ref[i, lo:lo + _C, :] = jnp.where(
                    rows < (nv - lo), rot[0:_C, :], 0.0)

            full = nv >= lo + _C
            tail = (nv > lo) & (nv < lo + _C)
            pl.when(full & jnp.logical_not(clamped))(
                lambda f=_fast: f(False))
            pl.when(tail & jnp.logical_not(clamped))(
                lambda f=_fast: f(True))
            pl.when((nv > lo) & clamped)(_slow)


@jax.jit
def kernel(startends, attention, context):
    att3 = attention.reshape(_BATCH, 1, _S)
    return pl.pallas_call(
        _kern,
        grid=(_NSTEP,),
        in_specs=[
            pl.BlockSpec((_BATCH, _N, 2), lambda g: (0, 0, 0)),
            pl.BlockSpec((_BATCH, 1, _S), lambda g: (0, 0, 0)),
            pl.BlockSpec(memory_space=pltpu.MemorySpace.HBM),
        ],
        out_specs=pl.BlockSpec((_BPS, _L, _D), lambda g: (g, 0, 0)),
        out_shape=jax.ShapeDtypeStruct((_BATCH, _L, _D), jnp.float32),
        scratch_shapes=[
            pltpu.VMEM((2, _BPS, _NCH, _W, _D), jnp.float32),
            pltpu.SMEM((2, _BPS, 2), jnp.int32),
            pltpu.SemaphoreType.DMA((2, _BPS)),
        ],
        compiler_params=pltpu.CompilerParams(
            dimension_semantics=("arbitrary",)),
    )(startends, att3, context)
